# baseline (speedup 1.0000x reference)
import functools

import jax
import jax.numpy as jnp
from jax import lax
from jax.experimental import pallas as pl
from jax.experimental.pallas import tpu as pltpu
from jax.experimental.pallas import tpu_sc as plsc

N_NODES = 10000
D_FEAT = 128
UNITS = 128
N_EDGES = 320000

NC, NS, L = 2, 16, 16
NW = NC * NS
E_W = N_EDGES // NW
K = 80
NCHUNK = E_W // K
ROWS_W = 624
TAIL_ROWS = N_NODES - NS * ROWS_W

_MESH = plsc.VectorSubcoreMesh(core_axis_name="c", subcore_axis_name="s")
_SC_PARAMS = pltpu.CompilerParams(needs_layout_passes=False)


@functools.partial(
    pl.kernel,
    out_type=[
        jax.ShapeDtypeStruct((NW, N_NODES), jnp.float32),
        jax.ShapeDtypeStruct((NW, N_NODES), jnp.float32),
    ],
    mesh=_MESH,
    compiler_params=_SC_PARAMS,
    scratch_types=[
        pltpu.VMEM((E_W,), jnp.int32),
        pltpu.VMEM((E_W,), jnp.int32),
        pltpu.VMEM((N_NODES,), jnp.float32),
        pltpu.VMEM((N_NODES,), jnp.float32),
    ],
)
def _degrees(src_hbm, dst_hbm, odeg_hbm, ideg_hbm, src_v, dst_v, oh_v, ih_v):
    c = lax.axis_index("c")
    s = lax.axis_index("s")
    wid = c * NS + s
    pltpu.sync_copy(src_hbm.at[wid], src_v)
    pltpu.sync_copy(dst_hbm.at[wid], dst_v)

    zero = jnp.zeros((L,), jnp.float32)

    def zbody(i, carry):
        off = pl.multiple_of(i * L, L)
        oh_v[pl.ds(off, L)] = zero
        ih_v[pl.ds(off, L)] = zero
        return carry

    lax.fori_loop(0, N_NODES // L, zbody, 0)

    ones = jnp.ones((L,), jnp.float32)

    def body(i, carry):
        off = pl.multiple_of(i * L, L)
        si = src_v[pl.ds(off, L)]
        di = dst_v[pl.ds(off, L)]
        plsc.addupdate_scatter(oh_v, [si], ones)
        plsc.addupdate_scatter(ih_v, [di], ones)
        return carry

    lax.fori_loop(0, E_W // L, body, 0)

    pltpu.sync_copy(oh_v, odeg_hbm.at[wid])
    pltpu.sync_copy(ih_v, ideg_hbm.at[wid])


@functools.partial(
    pl.kernel,
    out_type=jax.ShapeDtypeStruct((NC, N_NODES, UNITS), jnp.float32),
    mesh=_MESH,
    compiler_params=_SC_PARAMS,
    scratch_types=[
        [pltpu.VMEM((K,), jnp.int32)] * 3,
        [pltpu.VMEM((K,), jnp.int32)] * 3,
        [pltpu.VMEM((K, UNITS), jnp.float32)] * 3,
        pltpu.VMEM_SHARED((N_NODES, UNITS), jnp.float32),
        [pltpu.SemaphoreType.DMA] * 3,
        [pltpu.SemaphoreType.DMA] * 3,
    ],
)
def _segsum(y_hbm, src_hbm, dst_hbm, zeros_hbm, out_hbm,
            sidx, didx, rows, acc, gsem, isem):
    c = lax.axis_index("c")
    s = lax.axis_index("s")
    wid = c * NS + s

    r0 = s * ROWS_W
    pltpu.sync_copy(zeros_hbm, acc.at[pl.ds(r0, ROWS_W)])

    @pl.when(s == NS - 1)
    def _zero_tail():
        pltpu.sync_copy(
            zeros_hbm.at[pl.ds(0, TAIL_ROWS)],
            acc.at[pl.ds(NS * ROWS_W, TAIL_ROWS)],
        )

    plsc.subcore_barrier()

    def start_idx(i, p):
        pltpu.async_copy(src_hbm.at[wid, i], sidx[p], isem[p])
        pltpu.async_copy(dst_hbm.at[wid, i], didx[p], isem[p])

    def wait_idx(p):
        pltpu.make_async_copy(src_hbm.at[0, 0], sidx[p], isem[p]).wait()
        pltpu.make_async_copy(dst_hbm.at[0, 0], didx[p], isem[p]).wait()

    def start_gather(i, p):
        pltpu.async_copy(y_hbm.at[sidx[p]], rows[p], gsem[p])

    def wait_gather(p):
        pltpu.make_async_copy(y_hbm.at[pl.ds(0, K)], rows[p], gsem[p]).wait()

    start_idx(0, 0)
    start_idx(1, 1)
    start_idx(2, 2)
    wait_idx(0)
    start_gather(0, 0)
    wait_idx(1)
    start_gather(1, 1)

    def body(t, carry):
        for k in range(3):
            i = t * 3 + k
            p2 = (k + 2) % 3
            wait_idx(p2)
            start_gather(i + 2, p2)
            wait_gather(k)
            pltpu.sync_copy(rows[k], acc.at[didx[k]], add=True)

            @pl.when(i + 3 < NCHUNK)
            def _prefetch():
                start_idx(i + 3, k)

        return carry

    lax.fori_loop(0, NCHUNK // 3, body, 0)
    wait_gather(0)
    pltpu.sync_copy(rows[0], acc.at[didx[0]], add=True)
    wait_gather(1)
    pltpu.sync_copy(rows[1], acc.at[didx[1]], add=True)
    plsc.subcore_barrier()

    pltpu.sync_copy(acc.at[pl.ds(r0, ROWS_W)], out_hbm.at[c, pl.ds(r0, ROWS_W)])

    @pl.when(s == NS - 1)
    def _write_tail():
        pltpu.sync_copy(
            acc.at[pl.ds(NS * ROWS_W, TAIL_ROWS)],
            out_hbm.at[c, pl.ds(NS * ROWS_W, TAIL_ROWS)],
        )


_BLK = 1000
_GRID = N_NODES // _BLK


def _mm_body(od_ref, x_ref, w_ref, y_ref):
    deg = jnp.sum(od_ref[...], axis=1)
    sc = lax.rsqrt(jnp.maximum(deg, 1.0))
    y_ref[...] = jnp.dot(
        x_ref[...] * sc[:, None], w_ref[...], preferred_element_type=jnp.float32
    )


def _fin_body(p_ref, id_ref, o_ref):
    deg = jnp.sum(id_ref[...], axis=1)
    sn = lax.rsqrt(jnp.maximum(deg, 1.0))
    o_ref[...] = jnp.maximum((p_ref[0] + p_ref[1]) * sn[:, None], 0.0)


def kernel(x, edge_index, W):
    src = edge_index[:, 0].astype(jnp.int32)
    dst = edge_index[:, 1].astype(jnp.int32)
    src_w = src.reshape(NW, E_W)
    dst_w = dst.reshape(NW, E_W)
    src3 = src.reshape(NW, NCHUNK, K)
    dst3 = dst.reshape(NW, NCHUNK, K)

    odeg_p, ideg_p = _degrees(src_w, dst_w)

    y = pl.pallas_call(
        _mm_body,
        grid=(_GRID,),
        in_specs=[
            pl.BlockSpec((_BLK, NW), lambda i: (i, 0)),
            pl.BlockSpec((_BLK, D_FEAT), lambda i: (i, 0)),
            pl.BlockSpec((D_FEAT, UNITS), lambda i: (0, 0)),
        ],
        out_specs=pl.BlockSpec((_BLK, UNITS), lambda i: (i, 0)),
        out_shape=jax.ShapeDtypeStruct((N_NODES, UNITS), jnp.float32),
    )(odeg_p.T, x, W)

    zeros = jnp.zeros((ROWS_W, UNITS), jnp.float32)
    partials = _segsum(y, src3, dst3, zeros)

    out = pl.pallas_call(
        _fin_body,
        grid=(_GRID,),
        in_specs=[
            pl.BlockSpec((NC, _BLK, UNITS), lambda i: (0, i, 0)),
            pl.BlockSpec((_BLK, NW), lambda i: (i, 0)),
        ],
        out_specs=pl.BlockSpec((_BLK, UNITS), lambda i: (i, 0)),
        out_shape=jax.ShapeDtypeStruct((N_NODES, UNITS), jnp.float32),
    )(partials, ideg_p.T)
    return out


# speedup vs baseline: 1.1173x; 1.1173x over previous
import functools

import jax
import jax.numpy as jnp
from jax import lax
from jax.experimental import pallas as pl
from jax.experimental.pallas import tpu as pltpu
from jax.experimental.pallas import tpu_sc as plsc

N_NODES = 10000
D_FEAT = 128
UNITS = 128
N_EDGES = 320000

NC, NS, L = 2, 16, 16
NW = NC * NS
E_W = N_EDGES // NW
K = 80
NCHUNK = E_W // K
ROWS_W = 624
TAIL_ROWS = N_NODES - NS * ROWS_W

_MESH = plsc.VectorSubcoreMesh(core_axis_name="c", subcore_axis_name="s")
_SC_PARAMS = pltpu.CompilerParams(needs_layout_passes=False)


@functools.partial(
    pl.kernel,
    out_type=[
        jax.ShapeDtypeStruct((NW, N_NODES), jnp.float32),
        jax.ShapeDtypeStruct((NW, N_NODES), jnp.float32),
    ],
    mesh=_MESH,
    compiler_params=_SC_PARAMS,
    scratch_types=[
        pltpu.VMEM((E_W,), jnp.int32),
        pltpu.VMEM((E_W,), jnp.int32),
        pltpu.VMEM((N_NODES,), jnp.float32),
        pltpu.VMEM((N_NODES,), jnp.float32),
    ],
)
def _degrees(src_hbm, dst_hbm, odeg_hbm, ideg_hbm, src_v, dst_v, oh_v, ih_v):
    c = lax.axis_index("c")
    s = lax.axis_index("s")
    wid = c * NS + s
    pltpu.sync_copy(src_hbm.at[wid], src_v)
    pltpu.sync_copy(dst_hbm.at[wid], dst_v)

    zero = jnp.zeros((L,), jnp.float32)

    def zbody(i, carry):
        off = pl.multiple_of(i * L, L)
        oh_v[pl.ds(off, L)] = zero
        ih_v[pl.ds(off, L)] = zero
        return carry

    lax.fori_loop(0, N_NODES // L, zbody, 0)

    ones = jnp.ones((L,), jnp.float32)

    def body(i, carry):
        off = pl.multiple_of(i * L, L)
        si = src_v[pl.ds(off, L)]
        di = dst_v[pl.ds(off, L)]
        plsc.addupdate_scatter(oh_v, [si], ones)
        plsc.addupdate_scatter(ih_v, [di], ones)
        return carry

    lax.fori_loop(0, E_W // L, body, 0)

    pltpu.sync_copy(oh_v, odeg_hbm.at[wid])
    pltpu.sync_copy(ih_v, ideg_hbm.at[wid])


@functools.partial(
    pl.kernel,
    out_type=jax.ShapeDtypeStruct((NC, N_NODES, UNITS), jnp.float32),
    mesh=_MESH,
    compiler_params=_SC_PARAMS,
    scratch_types=[
        pltpu.VMEM((E_W,), jnp.int32),
        [pltpu.VMEM((K,), jnp.int32)] * 2,
        [pltpu.VMEM((K, UNITS), jnp.float32)] * 2,
        pltpu.VMEM_SHARED((N_NODES, UNITS), jnp.float32),
        [pltpu.SemaphoreType.DMA] * 2,
        [pltpu.SemaphoreType.DMA] * 2,
    ],
)
def _segsum(y_hbm, src_hbm, dst_hbm, zeros_hbm, out_hbm,
            sidx_v, didx, rows, acc, gsem, dsem):
    c = lax.axis_index("c")
    s = lax.axis_index("s")
    wid = c * NS + s

    r0 = s * ROWS_W
    pltpu.sync_copy(zeros_hbm, acc.at[pl.ds(r0, ROWS_W)])

    @pl.when(s == NS - 1)
    def _zero_tail():
        pltpu.sync_copy(
            zeros_hbm.at[pl.ds(0, TAIL_ROWS)],
            acc.at[pl.ds(NS * ROWS_W, TAIL_ROWS)],
        )

    plsc.subcore_barrier()

    pltpu.sync_copy(src_hbm.at[wid], sidx_v)

    def start_gather(i, p):
        pltpu.async_copy(y_hbm.at[sidx_v.at[pl.ds(i * K, K)]], rows[p], gsem[p])

    def wait_gather(p):
        pltpu.make_async_copy(y_hbm.at[pl.ds(0, K)], rows[p], gsem[p]).wait()

    def start_didx(i, p):
        pltpu.async_copy(dst_hbm.at[wid, i], didx[p], dsem[p])

    def wait_didx(p):
        pltpu.make_async_copy(dst_hbm.at[0, 0], didx[p], dsem[p]).wait()

    start_didx(0, 0)
    start_gather(0, 0)

    def body(j, carry):
        i = j * 2
        start_gather(i + 1, 1)
        start_didx(i + 1, 1)
        wait_didx(0)
        wait_gather(0)
        pltpu.sync_copy(rows[0], acc.at[didx[0]], add=True)
        start_gather(i + 2, 0)
        start_didx(i + 2, 0)
        wait_didx(1)
        wait_gather(1)
        pltpu.sync_copy(rows[1], acc.at[didx[1]], add=True)
        return carry

    lax.fori_loop(0, NCHUNK // 2, body, 0)
    wait_didx(0)
    wait_gather(0)
    pltpu.sync_copy(rows[0], acc.at[didx[0]], add=True)
    plsc.subcore_barrier()

    pltpu.sync_copy(acc.at[pl.ds(r0, ROWS_W)], out_hbm.at[c, pl.ds(r0, ROWS_W)])

    @pl.when(s == NS - 1)
    def _write_tail():
        pltpu.sync_copy(
            acc.at[pl.ds(NS * ROWS_W, TAIL_ROWS)],
            out_hbm.at[c, pl.ds(NS * ROWS_W, TAIL_ROWS)],
        )


_BLK = 1000
_GRID = N_NODES // _BLK


def _mm_body(od_ref, x_ref, w_ref, y_ref):
    deg = jnp.sum(od_ref[...], axis=1)
    sc = lax.rsqrt(jnp.maximum(deg, 1.0))
    y_ref[...] = jnp.dot(
        x_ref[...] * sc[:, None], w_ref[...], preferred_element_type=jnp.float32
    )


def _fin_body(p_ref, id_ref, o_ref):
    deg = jnp.sum(id_ref[...], axis=1)
    sn = lax.rsqrt(jnp.maximum(deg, 1.0))
    o_ref[...] = jnp.maximum((p_ref[0] + p_ref[1]) * sn[:, None], 0.0)


def kernel(x, edge_index, W):
    src_w = edge_index[:, 0].astype(jnp.int32).reshape(NW, E_W)
    dst_w = edge_index[:, 1].astype(jnp.int32).reshape(NW, E_W)
    dst3 = dst_w.reshape(NW, NCHUNK, K)

    odeg_p, ideg_p = _degrees(src_w, dst_w)

    y = pl.pallas_call(
        _mm_body,
        grid=(_GRID,),
        in_specs=[
            pl.BlockSpec((_BLK, NW), lambda i: (i, 0)),
            pl.BlockSpec((_BLK, D_FEAT), lambda i: (i, 0)),
            pl.BlockSpec((D_FEAT, UNITS), lambda i: (0, 0)),
        ],
        out_specs=pl.BlockSpec((_BLK, UNITS), lambda i: (i, 0)),
        out_shape=jax.ShapeDtypeStruct((N_NODES, UNITS), jnp.float32),
    )(odeg_p.T, x, W)

    zeros = jnp.zeros((ROWS_W, UNITS), jnp.float32)
    partials = _segsum(y, src_w, dst3, zeros)

    out = pl.pallas_call(
        _fin_body,
        grid=(_GRID,),
        in_specs=[
            pl.BlockSpec((NC, _BLK, UNITS), lambda i: (0, i, 0)),
            pl.BlockSpec((_BLK, NW), lambda i: (i, 0)),
        ],
        out_specs=pl.BlockSpec((_BLK, UNITS), lambda i: (i, 0)),
        out_shape=jax.ShapeDtypeStruct((N_NODES, UNITS), jnp.float32),
    )(partials, ideg_p.T)
    return out


# speedup vs baseline: 1.1437x; 1.0236x over previous
import functools

import jax
import jax.numpy as jnp
from jax import lax
from jax.experimental import pallas as pl
from jax.experimental.pallas import tpu as pltpu
from jax.experimental.pallas import tpu_sc as plsc

N_NODES = 10000
D_FEAT = 128
UNITS = 128
N_EDGES = 320000

NC, NS, L = 2, 16, 16
NW = NC * NS
E_W = N_EDGES // NW
K = 80
NCHUNK = E_W // K
ROWS_W = 624
TAIL_ROWS = N_NODES - NS * ROWS_W

_MESH = plsc.VectorSubcoreMesh(core_axis_name="c", subcore_axis_name="s")
_SC_PARAMS = pltpu.CompilerParams(needs_layout_passes=False)


@functools.partial(
    pl.kernel,
    out_type=[
        jax.ShapeDtypeStruct((NW, N_NODES), jnp.float32),
        jax.ShapeDtypeStruct((NW, N_NODES), jnp.float32),
    ],
    mesh=_MESH,
    compiler_params=_SC_PARAMS,
    scratch_types=[
        pltpu.VMEM((E_W,), jnp.int32),
        pltpu.VMEM((E_W,), jnp.int32),
        pltpu.VMEM((N_NODES,), jnp.float32),
        pltpu.VMEM((N_NODES,), jnp.float32),
    ],
)
def _degrees(src_hbm, dst_hbm, odeg_hbm, ideg_hbm, src_v, dst_v, oh_v, ih_v):
    c = lax.axis_index("c")
    s = lax.axis_index("s")
    wid = c * NS + s
    pltpu.sync_copy(src_hbm.at[wid], src_v)
    pltpu.sync_copy(dst_hbm.at[wid], dst_v)

    zero = jnp.zeros((L,), jnp.float32)

    def zbody(i, carry):
        off = pl.multiple_of(i * L, L)
        oh_v[pl.ds(off, L)] = zero
        ih_v[pl.ds(off, L)] = zero
        return carry

    lax.fori_loop(0, N_NODES // L, zbody, 0)

    ones = jnp.ones((L,), jnp.float32)

    def body(i, carry):
        off = pl.multiple_of(i * L, L)
        si = src_v[pl.ds(off, L)]
        di = dst_v[pl.ds(off, L)]
        plsc.addupdate_scatter(oh_v, [si], ones)
        plsc.addupdate_scatter(ih_v, [di], ones)
        return carry

    lax.fori_loop(0, E_W // L, body, 0)

    pltpu.sync_copy(oh_v, odeg_hbm.at[wid])
    pltpu.sync_copy(ih_v, ideg_hbm.at[wid])


@functools.partial(
    pl.kernel,
    out_type=jax.ShapeDtypeStruct((NC, N_NODES, UNITS), jnp.float32),
    mesh=_MESH,
    compiler_params=_SC_PARAMS,
    scratch_types=[
        pltpu.VMEM((E_W,), jnp.int32),
        [pltpu.VMEM((K,), jnp.int32)] * 2,
        [pltpu.VMEM((K, UNITS), jnp.float32)] * 2,
        pltpu.VMEM_SHARED((N_NODES, UNITS), jnp.float32),
        [pltpu.SemaphoreType.DMA] * 2,
        [pltpu.SemaphoreType.DMA] * 2,
    ],
)
def _segsum(y_hbm, src_hbm, dst_hbm, zeros_hbm, out_hbm,
            sidx_v, didx, rows, acc, gsem, dsem):
    c = lax.axis_index("c")
    s = lax.axis_index("s")
    wid = c * NS + s

    r0 = s * ROWS_W
    pltpu.sync_copy(zeros_hbm, acc.at[pl.ds(r0, ROWS_W)])

    @pl.when(s == NS - 1)
    def _zero_tail():
        pltpu.sync_copy(
            zeros_hbm.at[pl.ds(0, TAIL_ROWS)],
            acc.at[pl.ds(NS * ROWS_W, TAIL_ROWS)],
        )

    plsc.subcore_barrier()

    pltpu.sync_copy(src_hbm.at[wid], sidx_v)

    H = K // 2

    def start_gather(i, p):
        pltpu.async_copy(
            y_hbm.at[sidx_v.at[pl.ds(i * K, H)]], rows[p].at[pl.ds(0, H)], gsem[p]
        )
        pltpu.async_copy(
            y_hbm.at[sidx_v.at[pl.ds(i * K + H, H)]], rows[p].at[pl.ds(H, H)], gsem[p]
        )

    def wait_gather(p):
        pltpu.make_async_copy(y_hbm.at[pl.ds(0, H)], rows[p].at[pl.ds(0, H)], gsem[p]).wait()
        pltpu.make_async_copy(y_hbm.at[pl.ds(0, H)], rows[p].at[pl.ds(H, H)], gsem[p]).wait()

    def start_didx(i, p):
        pltpu.async_copy(dst_hbm.at[wid, i], didx[p], dsem[p])

    def wait_didx(p):
        pltpu.make_async_copy(dst_hbm.at[0, 0], didx[p], dsem[p]).wait()

    start_didx(0, 0)
    start_gather(0, 0)

    def body(j, carry):
        i = j * 2
        start_gather(i + 1, 1)
        start_didx(i + 1, 1)
        wait_didx(0)
        wait_gather(0)
        pltpu.sync_copy(rows[0], acc.at[didx[0]], add=True)
        start_gather(i + 2, 0)
        start_didx(i + 2, 0)
        wait_didx(1)
        wait_gather(1)
        pltpu.sync_copy(rows[1], acc.at[didx[1]], add=True)
        return carry

    lax.fori_loop(0, NCHUNK // 2, body, 0)
    wait_didx(0)
    wait_gather(0)
    pltpu.sync_copy(rows[0], acc.at[didx[0]], add=True)
    plsc.subcore_barrier()

    pltpu.sync_copy(acc.at[pl.ds(r0, ROWS_W)], out_hbm.at[c, pl.ds(r0, ROWS_W)])

    @pl.when(s == NS - 1)
    def _write_tail():
        pltpu.sync_copy(
            acc.at[pl.ds(NS * ROWS_W, TAIL_ROWS)],
            out_hbm.at[c, pl.ds(NS * ROWS_W, TAIL_ROWS)],
        )


_BLK = 2000
_GRID = N_NODES // _BLK


def _mm_body(od_ref, x_ref, w_ref, y_ref):
    deg = jnp.sum(od_ref[...], axis=1)
    sc = lax.rsqrt(jnp.maximum(deg, 1.0))
    y_ref[...] = jnp.dot(
        x_ref[...] * sc[:, None], w_ref[...], preferred_element_type=jnp.float32
    )


def _fin_body(p_ref, id_ref, o_ref):
    deg = jnp.sum(id_ref[...], axis=1)
    sn = lax.rsqrt(jnp.maximum(deg, 1.0))
    o_ref[...] = jnp.maximum((p_ref[0] + p_ref[1]) * sn[:, None], 0.0)


def kernel(x, edge_index, W):
    src_w = edge_index[:, 0].astype(jnp.int32).reshape(NW, E_W)
    dst_w = edge_index[:, 1].astype(jnp.int32).reshape(NW, E_W)
    dst3 = dst_w.reshape(NW, NCHUNK, K)

    odeg_p, ideg_p = _degrees(src_w, dst_w)

    y = pl.pallas_call(
        _mm_body,
        grid=(_GRID,),
        in_specs=[
            pl.BlockSpec((_BLK, NW), lambda i: (i, 0)),
            pl.BlockSpec((_BLK, D_FEAT), lambda i: (i, 0)),
            pl.BlockSpec((D_FEAT, UNITS), lambda i: (0, 0)),
        ],
        out_specs=pl.BlockSpec((_BLK, UNITS), lambda i: (i, 0)),
        out_shape=jax.ShapeDtypeStruct((N_NODES, UNITS), jnp.float32),
    )(odeg_p.T, x, W)

    zeros = jnp.zeros((ROWS_W, UNITS), jnp.float32)
    partials = _segsum(y, src_w, dst3, zeros)

    out = pl.pallas_call(
        _fin_body,
        grid=(_GRID,),
        in_specs=[
            pl.BlockSpec((NC, _BLK, UNITS), lambda i: (0, i, 0)),
            pl.BlockSpec((_BLK, NW), lambda i: (i, 0)),
        ],
        out_specs=pl.BlockSpec((_BLK, UNITS), lambda i: (i, 0)),
        out_shape=jax.ShapeDtypeStruct((N_NODES, UNITS), jnp.float32),
    )(partials, ideg_p.T)
    return out


# speedup vs baseline: 1.2231x; 1.0694x over previous
import functools

import jax
import jax.numpy as jnp
from jax import lax
from jax.experimental import pallas as pl
from jax.experimental.pallas import tpu as pltpu
from jax.experimental.pallas import tpu_sc as plsc

N_NODES = 10000
D_FEAT = 128
UNITS = 128
N_EDGES = 320000

NC, NS, L = 2, 16, 16
NW = NC * NS
E_W = N_EDGES // NW
K = 80
NCHUNK = E_W // K
ROWS_W = 624
TAIL_ROWS = N_NODES - NS * ROWS_W

_MESH = plsc.VectorSubcoreMesh(core_axis_name="c", subcore_axis_name="s")
_SC_PARAMS = pltpu.CompilerParams(needs_layout_passes=False)


@functools.partial(
    pl.kernel,
    out_type=[
        jax.ShapeDtypeStruct((NW, N_NODES), jnp.float32),
        jax.ShapeDtypeStruct((NW, N_NODES), jnp.float32),
        jax.ShapeDtypeStruct((NW, NCHUNK, K), jnp.int32),
    ],
    mesh=_MESH,
    compiler_params=_SC_PARAMS,
    scratch_types=[
        pltpu.VMEM((E_W,), jnp.int32),
        pltpu.VMEM((E_W,), jnp.int32),
        pltpu.VMEM((NCHUNK, K), jnp.int32),
        pltpu.VMEM((N_NODES,), jnp.float32),
        pltpu.VMEM((N_NODES,), jnp.float32),
    ],
)
def _degrees(src_hbm, dst_hbm, odeg_hbm, ideg_hbm, dstc_hbm,
             src_v, dst_v, dstc_v, oh_v, ih_v):
    c = lax.axis_index("c")
    s = lax.axis_index("s")
    wid = c * NS + s
    pltpu.sync_copy(src_hbm.at[wid], src_v)
    pltpu.sync_copy(dst_hbm.at[wid], dst_v)

    zero = jnp.zeros((L,), jnp.float32)

    def zbody(i, carry):
        off = pl.multiple_of(i * L, L)
        oh_v[pl.ds(off, L)] = zero
        ih_v[pl.ds(off, L)] = zero
        return carry

    lax.fori_loop(0, N_NODES // L, zbody, 0)

    ones = jnp.ones((L,), jnp.float32)
    GP = K // L

    def body(i, carry):
        off = pl.multiple_of(i * L, L)
        si = src_v[pl.ds(off, L)]
        di = dst_v[pl.ds(off, L)]
        plsc.addupdate_scatter(oh_v, [si], ones)
        plsc.addupdate_scatter(ih_v, [di], ones)
        dstc_v[i // GP, pl.ds((i % GP) * L, L)] = di
        return carry

    lax.fori_loop(0, E_W // L, body, 0)

    pltpu.sync_copy(oh_v, odeg_hbm.at[wid])
    pltpu.sync_copy(ih_v, ideg_hbm.at[wid])
    pltpu.sync_copy(dstc_v, dstc_hbm.at[wid])


@functools.partial(
    pl.kernel,
    out_type=jax.ShapeDtypeStruct((NC, N_NODES, UNITS), jnp.float32),
    mesh=_MESH,
    compiler_params=_SC_PARAMS,
    scratch_types=[
        pltpu.VMEM((E_W,), jnp.int32),
        [pltpu.VMEM((K,), jnp.int32)] * 2,
        [pltpu.VMEM((K, UNITS), jnp.float32)] * 2,
        pltpu.VMEM_SHARED((N_NODES, UNITS), jnp.float32),
        [pltpu.SemaphoreType.DMA] * 2,
        [pltpu.SemaphoreType.DMA] * 2,
    ],
)
def _segsum(y_hbm, src_hbm, dst_hbm, out_hbm,
            sidx_v, didx, rows, acc, gsem, dsem):
    c = lax.axis_index("c")
    s = lax.axis_index("s")
    wid = c * NS + s

    zero = jnp.zeros((L,), jnp.float32)

    def _zrow(i, carry):
        for j in range(UNITS // L):
            rows[0][i, pl.ds(j * L, L)] = zero
        return carry

    lax.fori_loop(0, K, _zrow, 0)
    r0 = s * ROWS_W
    for t in range(ROWS_W // K):
        pltpu.sync_copy(rows[0], acc.at[pl.ds(r0 + t * K, K)])
    rem = ROWS_W % K
    pltpu.sync_copy(
        rows[0].at[pl.ds(0, rem)],
        acc.at[pl.ds(r0 + (ROWS_W // K) * K, rem)],
    )

    @pl.when(s == NS - 1)
    def _zero_tail():
        pltpu.sync_copy(
            rows[0].at[pl.ds(0, TAIL_ROWS)],
            acc.at[pl.ds(NS * ROWS_W, TAIL_ROWS)],
        )

    plsc.subcore_barrier()

    pltpu.sync_copy(src_hbm.at[wid], sidx_v)

    H = K // 2

    def start_gather(i, p):
        pltpu.async_copy(
            y_hbm.at[sidx_v.at[pl.ds(i * K, H)]], rows[p].at[pl.ds(0, H)], gsem[p]
        )
        pltpu.async_copy(
            y_hbm.at[sidx_v.at[pl.ds(i * K + H, H)]], rows[p].at[pl.ds(H, H)], gsem[p]
        )

    def wait_gather(p):
        pltpu.make_async_copy(y_hbm.at[pl.ds(0, H)], rows[p].at[pl.ds(0, H)], gsem[p]).wait()
        pltpu.make_async_copy(y_hbm.at[pl.ds(0, H)], rows[p].at[pl.ds(H, H)], gsem[p]).wait()

    def start_didx(i, p):
        pltpu.async_copy(dst_hbm.at[wid, i], didx[p], dsem[p])

    def wait_didx(p):
        pltpu.make_async_copy(dst_hbm.at[0, 0], didx[p], dsem[p]).wait()

    start_didx(0, 0)
    start_gather(0, 0)

    def body(j, carry):
        i = j * 2
        start_gather(i + 1, 1)
        start_didx(i + 1, 1)
        wait_didx(0)
        wait_gather(0)
        pltpu.sync_copy(rows[0], acc.at[didx[0]], add=True)
        start_gather(i + 2, 0)
        start_didx(i + 2, 0)
        wait_didx(1)
        wait_gather(1)
        pltpu.sync_copy(rows[1], acc.at[didx[1]], add=True)
        return carry

    lax.fori_loop(0, NCHUNK // 2, body, 0)
    wait_didx(0)
    wait_gather(0)
    pltpu.sync_copy(rows[0], acc.at[didx[0]], add=True)
    plsc.subcore_barrier()

    pltpu.sync_copy(acc.at[pl.ds(r0, ROWS_W)], out_hbm.at[c, pl.ds(r0, ROWS_W)])

    @pl.when(s == NS - 1)
    def _write_tail():
        pltpu.sync_copy(
            acc.at[pl.ds(NS * ROWS_W, TAIL_ROWS)],
            out_hbm.at[c, pl.ds(NS * ROWS_W, TAIL_ROWS)],
        )


_BLK = 2000
_GRID = N_NODES // _BLK


def _mm_body(od_ref, x_ref, w_ref, y_ref):
    deg = jnp.sum(od_ref[...], axis=0)
    sc = lax.rsqrt(jnp.maximum(deg, 1.0))
    y_ref[...] = jnp.dot(
        x_ref[...] * sc[:, None], w_ref[...], preferred_element_type=jnp.float32
    )


def _fin_body(p_ref, id_ref, o_ref):
    deg = jnp.sum(id_ref[...], axis=1)
    sn = lax.rsqrt(jnp.maximum(deg, 1.0))
    o_ref[...] = jnp.maximum((p_ref[0] + p_ref[1]) * sn[:, None], 0.0)


def kernel(x, edge_index, W):
    src_w = edge_index[:, 0].astype(jnp.int32).reshape(NW, E_W)
    dst_w = edge_index[:, 1].astype(jnp.int32).reshape(NW, E_W)

    odeg_p, ideg_p, dst_c = _degrees(src_w, dst_w)

    y = pl.pallas_call(
        _mm_body,
        out_shape=jax.ShapeDtypeStruct((N_NODES, UNITS), jnp.float32),
    )(odeg_p, x, W)

    partials = _segsum(y, src_w, dst_c)

    out = pl.pallas_call(
        _fin_body,
        grid=(_GRID,),
        in_specs=[
            pl.BlockSpec((NC, _BLK, UNITS), lambda i: (0, i, 0)),
            pl.BlockSpec((_BLK, NW), lambda i: (i, 0)),
        ],
        out_specs=pl.BlockSpec((_BLK, UNITS), lambda i: (i, 0)),
        out_shape=jax.ShapeDtypeStruct((N_NODES, UNITS), jnp.float32),
    )(partials, ideg_p.T)
    return out


# speedup vs baseline: 1.2342x; 1.0090x over previous
import functools

import jax
import jax.numpy as jnp
from jax import lax
from jax.experimental import pallas as pl
from jax.experimental.pallas import tpu as pltpu
from jax.experimental.pallas import tpu_sc as plsc

N_NODES = 10000
D_FEAT = 128
UNITS = 128
N_EDGES = 320000

NC, NS, L = 2, 16, 16
NW = NC * NS
E_W = N_EDGES // NW
K = 80
NCHUNK = E_W // K
ROWS_W = 624
TAIL_ROWS = N_NODES - NS * ROWS_W

_MESH = plsc.VectorSubcoreMesh(core_axis_name="c", subcore_axis_name="s")
_SC_PARAMS = pltpu.CompilerParams(needs_layout_passes=False)


@functools.partial(
    pl.kernel,
    out_type=[
        jax.ShapeDtypeStruct((NW, N_NODES), jnp.float32),
        jax.ShapeDtypeStruct((NW, N_NODES), jnp.float32),
        jax.ShapeDtypeStruct((NW, NCHUNK, K), jnp.int32),
    ],
    mesh=_MESH,
    compiler_params=_SC_PARAMS,
    scratch_types=[
        pltpu.VMEM((E_W,), jnp.int32),
        pltpu.VMEM((E_W,), jnp.int32),
        pltpu.VMEM((NCHUNK, K), jnp.int32),
        pltpu.VMEM((N_NODES,), jnp.float32),
        pltpu.VMEM((N_NODES,), jnp.float32),
    ],
)
def _degrees(src_hbm, dst_hbm, odeg_hbm, ideg_hbm, dstc_hbm,
             src_v, dst_v, dstc_v, oh_v, ih_v):
    c = lax.axis_index("c")
    s = lax.axis_index("s")
    wid = c * NS + s
    pltpu.sync_copy(src_hbm.at[wid], src_v)
    pltpu.sync_copy(dst_hbm.at[wid], dst_v)

    zero = jnp.zeros((L,), jnp.float32)

    def zbody(i, carry):
        off = pl.multiple_of(i * L, L)
        oh_v[pl.ds(off, L)] = zero
        ih_v[pl.ds(off, L)] = zero
        return carry

    lax.fori_loop(0, N_NODES // L, zbody, 0)

    ones = jnp.ones((L,), jnp.float32)
    GP = K // L

    def body(i, carry):
        off = pl.multiple_of(i * L, L)
        si = src_v[pl.ds(off, L)]
        di = dst_v[pl.ds(off, L)]
        plsc.addupdate_scatter(oh_v, [si], ones)
        plsc.addupdate_scatter(ih_v, [di], ones)
        dstc_v[i // GP, pl.ds((i % GP) * L, L)] = di
        return carry

    lax.fori_loop(0, E_W // L, body, 0)

    pltpu.sync_copy(oh_v, odeg_hbm.at[wid])
    pltpu.sync_copy(ih_v, ideg_hbm.at[wid])
    pltpu.sync_copy(dstc_v, dstc_hbm.at[wid])


@functools.partial(
    pl.kernel,
    out_type=jax.ShapeDtypeStruct((NC, N_NODES, UNITS), jnp.float32),
    mesh=_MESH,
    compiler_params=_SC_PARAMS,
    scratch_types=[
        pltpu.VMEM((E_W,), jnp.int32),
        [pltpu.VMEM((K,), jnp.int32)] * 2,
        [pltpu.VMEM((K, UNITS), jnp.float32)] * 2,
        pltpu.VMEM_SHARED((N_NODES, UNITS), jnp.float32),
        [pltpu.SemaphoreType.DMA] * 2,
        [pltpu.SemaphoreType.DMA] * 2,
        pltpu.SemaphoreType.DMA,
    ],
)
def _segsum(y_hbm, src_hbm, dst_hbm, out_hbm,
            sidx_v, didx, rows, acc, gsem, dsem, psem):
    c = lax.axis_index("c")
    s = lax.axis_index("s")
    wid = c * NS + s
    pltpu.async_copy(src_hbm.at[wid], sidx_v, psem)

    zero = jnp.zeros((L,), jnp.float32)

    def _zrow(i, carry):
        for j in range(UNITS // L):
            rows[0][i, pl.ds(j * L, L)] = zero
        return carry

    lax.fori_loop(0, K, _zrow, 0)
    r0 = s * ROWS_W
    for t in range(ROWS_W // K):
        pltpu.sync_copy(rows[0], acc.at[pl.ds(r0 + t * K, K)])
    rem = ROWS_W % K
    pltpu.sync_copy(
        rows[0].at[pl.ds(0, rem)],
        acc.at[pl.ds(r0 + (ROWS_W // K) * K, rem)],
    )

    @pl.when(s == NS - 1)
    def _zero_tail():
        pltpu.sync_copy(
            rows[0].at[pl.ds(0, TAIL_ROWS)],
            acc.at[pl.ds(NS * ROWS_W, TAIL_ROWS)],
        )

    plsc.subcore_barrier()
    pltpu.make_async_copy(src_hbm.at[0], sidx_v, psem).wait()

    NSTREAM = 5
    Q = K // NSTREAM

    def start_gather(i, p):
        for q in range(NSTREAM):
            pltpu.async_copy(
                y_hbm.at[sidx_v.at[pl.ds(i * K + q * Q, Q)]],
                rows[p].at[pl.ds(q * Q, Q)],
                gsem[p],
            )

    def wait_gather(p):
        for q in range(NSTREAM):
            pltpu.make_async_copy(
                y_hbm.at[pl.ds(0, Q)], rows[p].at[pl.ds(q * Q, Q)], gsem[p]
            ).wait()

    def start_didx(i, p):
        pltpu.async_copy(dst_hbm.at[wid, i], didx[p], dsem[p])

    def wait_didx(p):
        pltpu.make_async_copy(dst_hbm.at[0, 0], didx[p], dsem[p]).wait()

    start_didx(0, 0)
    start_gather(0, 0)

    def body(j, carry):
        i = j * 2
        start_gather(i + 1, 1)
        start_didx(i + 1, 1)
        wait_didx(0)
        wait_gather(0)
        pltpu.sync_copy(rows[0], acc.at[didx[0]], add=True)
        start_gather(i + 2, 0)
        start_didx(i + 2, 0)
        wait_didx(1)
        wait_gather(1)
        pltpu.sync_copy(rows[1], acc.at[didx[1]], add=True)
        return carry

    lax.fori_loop(0, NCHUNK // 2, body, 0)
    wait_didx(0)
    wait_gather(0)
    pltpu.sync_copy(rows[0], acc.at[didx[0]], add=True)
    plsc.subcore_barrier()

    pltpu.sync_copy(acc.at[pl.ds(r0, ROWS_W)], out_hbm.at[c, pl.ds(r0, ROWS_W)])

    @pl.when(s == NS - 1)
    def _write_tail():
        pltpu.sync_copy(
            acc.at[pl.ds(NS * ROWS_W, TAIL_ROWS)],
            out_hbm.at[c, pl.ds(NS * ROWS_W, TAIL_ROWS)],
        )


_BLK = 2000
_GRID = N_NODES // _BLK


def _mm_body(od_ref, x_ref, w_ref, y_ref):
    deg = jnp.sum(od_ref[...], axis=0)
    sc = lax.rsqrt(jnp.maximum(deg, 1.0))
    y_ref[...] = jnp.dot(
        x_ref[...] * sc[:, None], w_ref[...], preferred_element_type=jnp.float32
    )


def _fin_body(p_ref, id_ref, o_ref):
    deg = jnp.sum(id_ref[...], axis=1)
    sn = lax.rsqrt(jnp.maximum(deg, 1.0))
    o_ref[...] = jnp.maximum((p_ref[0] + p_ref[1]) * sn[:, None], 0.0)


def kernel(x, edge_index, W):
    src_w = edge_index[:, 0].astype(jnp.int32).reshape(NW, E_W)
    dst_w = edge_index[:, 1].astype(jnp.int32).reshape(NW, E_W)

    odeg_p, ideg_p, dst_c = _degrees(src_w, dst_w)

    y = pl.pallas_call(
        _mm_body,
        out_shape=jax.ShapeDtypeStruct((N_NODES, UNITS), jnp.float32),
    )(odeg_p, x, W)

    partials = _segsum(y, src_w, dst_c)

    out = pl.pallas_call(
        _fin_body,
        grid=(_GRID,),
        in_specs=[
            pl.BlockSpec((NC, _BLK, UNITS), lambda i: (0, i, 0)),
            pl.BlockSpec((_BLK, NW), lambda i: (i, 0)),
        ],
        out_specs=pl.BlockSpec((_BLK, UNITS), lambda i: (i, 0)),
        out_shape=jax.ShapeDtypeStruct((N_NODES, UNITS), jnp.float32),
    )(partials, ideg_p.T)
    return out
